# Initial kernel scaffold; baseline (speedup 1.0000x reference)
import functools

import jax
import jax.numpy as jnp
from jax import lax
from jax.experimental import pallas as pl
from jax.experimental.pallas import tpu as pltpu

D = 64
_ROW_BLK = 2000


def _norm_kernel(fraw_ref, out_ref):
    x = fraw_ref[...]
    n = jnp.sqrt(jnp.sum(x * x, axis=1, keepdims=True))
    out_ref[...] = x / jnp.maximum(n, 1e-12)


def _combine_kernel(f0_ref, f1_ref, f2raw_ref, out_ref):
    x = f2raw_ref[...]
    n = jnp.sqrt(jnp.sum(x * x, axis=1, keepdims=True))
    f2 = x / jnp.maximum(n, 1e-12)
    out_ref[...] = (f0_ref[...] + f1_ref[...] + f2) * (1.0 / 3.0)


def _rows_spec():
    return pl.BlockSpec((_ROW_BLK, D), lambda i: (i, 0))


def _normalize(fraw):
    n = fraw.shape[0]
    return pl.pallas_call(
        _norm_kernel,
        grid=(n // _ROW_BLK,),
        in_specs=[_rows_spec()],
        out_specs=_rows_spec(),
        out_shape=jax.ShapeDtypeStruct((n, D), jnp.float32),
    )(fraw)


def _combine(f0, f1, f2raw):
    n = f0.shape[0]
    return pl.pallas_call(
        _combine_kernel,
        grid=(n // _ROW_BLK,),
        in_specs=[_rows_spec(), _rows_spec(), _rows_spec()],
        out_specs=_rows_spec(),
        out_shape=jax.ShapeDtypeStruct((n, D), jnp.float32),
    )(f0, f1, f2raw)


def _spmm(f, src, dst, w, n_total):
    return jax.ops.segment_sum(f[src] * w[:, None], dst, num_segments=n_total)


def _propagate(A, B, src, dst, w):
    nA = A.shape[0]
    n_total = nA + B.shape[0]
    f0 = jnp.concatenate([A, B], axis=0)
    f1raw = _spmm(f0, src, dst, w, n_total)
    f1 = _normalize(f1raw)
    f2raw = _spmm(f1, src, dst, w, n_total)
    agg = _combine(f0, f1, f2raw)
    return agg[:nA], agg[nA:]


def kernel(users_feature, items_feature, bundles_feature, ui_src, ui_dst, ui_w, bi_src, bi_dst, bi_w, ub_src, ub_dst, ub_w):
    UI_u, UI_i = _propagate(users_feature, items_feature, ui_src, ui_dst, ui_w)
    BI_b, BI_i = _propagate(bundles_feature, items_feature, bi_src, bi_dst, bi_w)
    UB_u, UB_b = _propagate(users_feature, bundles_feature, ub_src, ub_dst, ub_w)
    return (UI_u, UB_u, BI_b, BI_i, UB_b, UI_i)


# speedup vs baseline: 1.0007x; 1.0007x over previous
import functools

import jax
import jax.numpy as jnp
from jax import lax
from jax.experimental import pallas as pl
from jax.experimental.pallas import tpu as pltpu

D = 64
_ROW_BLK = 2000


def _l2n(x):
    n = jnp.sqrt(jnp.sum(x * x, axis=1, keepdims=True))
    return x / jnp.maximum(n, 1e-12)


def _combine_kernel(f0_ref, f1raw_ref, f2raw_ref, out_ref):
    out_ref[...] = (f0_ref[...] + _l2n(f1raw_ref[...]) + _l2n(f2raw_ref[...])) * (1.0 / 3.0)


def _rows_spec():
    return pl.BlockSpec((_ROW_BLK, D), lambda i: (i, 0))


def _combine(f0, f1raw, f2raw):
    n = f0.shape[0]
    return pl.pallas_call(
        _combine_kernel,
        grid=(n // _ROW_BLK,),
        in_specs=[_rows_spec(), _rows_spec(), _rows_spec()],
        out_specs=_rows_spec(),
        out_shape=jax.ShapeDtypeStruct((n, D), jnp.float32),
    )(f0, f1raw, f2raw)


def _spmm(f, src, dst, w, n_total):
    return jax.ops.segment_sum(f[src] * w[:, None], dst, num_segments=n_total)


def _propagate(A, B, src, dst, w):
    nA = A.shape[0]
    n_total = nA + B.shape[0]
    f0 = jnp.concatenate([A, B], axis=0)
    f1raw = _spmm(f0, src, dst, w, n_total)
    f2raw = _spmm(f1raw, src, dst, w, n_total)
    agg = _combine(f0, f1raw, f2raw)
    return agg[:nA], agg[nA:]


def kernel(users_feature, items_feature, bundles_feature, ui_src, ui_dst, ui_w, bi_src, bi_dst, bi_w, ub_src, ub_dst, ub_w):
    UI_u, UI_i = _propagate(users_feature, items_feature, ui_src, ui_dst, ui_w)
    BI_b, BI_i = _propagate(bundles_feature, items_feature, bi_src, bi_dst, bi_w)
    UB_u, UB_b = _propagate(users_feature, bundles_feature, ub_src, ub_dst, ub_w)
    return (UI_u, UB_u, BI_b, BI_i, UB_b, UI_i)


# speedup vs baseline: 3.0571x; 3.0549x over previous
import functools

import jax
import jax.numpy as jnp
from jax import lax
from jax.experimental import pallas as pl
from jax.experimental.pallas import tpu as pltpu
from jax.experimental.pallas import tpu_sc as plsc

D = 64
_ROW_BLK = 2000
_C = 128
_NT = 16
_NC = 2
_EDGE_ALIGN = _NT * _C
_ACC_ROWS = 25088
_ZROWS = 1600


_GDN = lax.GatherDimensionNumbers(
    offset_dims=(), collapsed_slice_dims=(0,), start_index_map=(0,))


def _bcast_lane(v16, j):
    idx = jnp.full((16, 1), j, jnp.int32)
    return lax.gather(v16, idx, _GDN, slice_sizes=(1,),
                      mode=lax.GatherScatterMode.PROMISE_IN_BOUNDS)


def _l2n(x):
    n = jnp.sqrt(jnp.sum(x * x, axis=1, keepdims=True))
    return x / jnp.maximum(n, 1e-12)


def _combine_kernel(f0_ref, f1raw_ref, f2raw_ref, out_ref):
    out_ref[...] = (f0_ref[...] + _l2n(f1raw_ref[...]) + _l2n(f2raw_ref[...])) * (1.0 / 3.0)


def _rows_spec():
    return pl.BlockSpec((_ROW_BLK, D), lambda i: (i, 0))


def _combine(f0, f1raw, f2raw):
    n = f0.shape[0]
    return pl.pallas_call(
        _combine_kernel,
        grid=(n // _ROW_BLK,),
        in_specs=[_rows_spec(), _rows_spec(), _rows_spec()],
        out_specs=_rows_spec(),
        out_shape=jax.ShapeDtypeStruct((n, D), jnp.float32),
    )(f0, f1raw, f2raw)


@functools.lru_cache(maxsize=None)
def _make_spmm(n, m, ehp):
    n_total = n + m
    nct = ehp // _EDGE_ALIGN
    mesh = plsc.VectorSubcoreMesh(core_axis_name="c", subcore_axis_name="s")

    @functools.partial(
        pl.kernel,
        mesh=mesh,
        compiler_params=pltpu.CompilerParams(use_tc_tiling_on_sc=False),
        out_type=jax.ShapeDtypeStruct((n_total, D), jnp.float32),
        scratch_types=[
            pltpu.VMEM((_C,), jnp.int32),
            pltpu.VMEM((_C,), jnp.int32),
            pltpu.VMEM((_C,), jnp.float32),
            pltpu.VMEM((_C,), jnp.int32),
            pltpu.VMEM((_C, D), jnp.float32),
            pltpu.VMEM_SHARED((_ACC_ROWS, D), jnp.float32),
            pltpu.SemaphoreType.DMA,
        ],
    )
    def spmm(f_hbm, src2, dst2, w2, zeros_hbm, out_hbm,
             srcb, dstb, wb, idxb, rows, acc, sem):
        c = lax.axis_index("c")
        s = lax.axis_index("s")
        per_tile = ehp // _NT

        for h, r_rows, obase in ((1, n // _NC, 0), (0, m // _NC, n)):
            stripe = ((r_rows + 16 + _NT * 8 - 1) // (_NT * 8)) * 8
            pltpu.sync_copy(zeros_hbm.at[pl.ds(0, stripe)],
                            acc.at[pl.ds(s * stripe, stripe)])
            plsc.subcore_barrier()
            rowbase = obase + c * r_rows

            def chunk_body(k, _, h=h, r_rows=r_rows, rowbase=rowbase):
                off = s * per_tile + k * _C
                pltpu.sync_copy(src2.at[h, pl.ds(off, _C)], srcb)
                pltpu.sync_copy(dst2.at[h, pl.ds(off, _C)], dstb)
                pltpu.sync_copy(w2.at[h, pl.ds(off, _C)], wb)
                pltpu.async_copy(f_hbm.at[srcb], rows, sem).wait()
                for g in range(_C // 16):
                    w16 = wb[pl.ds(g * 16, 16)]
                    d16 = dstb[pl.ds(g * 16, 16)]
                    rel = d16 - rowbase
                    ok = (rel >= 0) & (rel < r_rows)
                    idxb[pl.ds(g * 16, 16)] = jnp.where(ok, rel, r_rows)
                    for j in range(16):
                        i = g * 16 + j
                        wr = _bcast_lane(w16, j)
                        for q in range(4):
                            rows[i, pl.ds(q * 16, 16)] = rows[i, pl.ds(q * 16, 16)] * wr
                pltpu.sync_copy(rows, acc.at[idxb], add=True)
                return 0

            lax.fori_loop(0, nct, chunk_body, 0)
            plsc.subcore_barrier()

            chunks8 = r_rows // 8
            nflush = (chunks8 + _NT - 1) // _NT

            def flush_body(kf, _, rowbase=rowbase, chunks8=chunks8):
                j = kf * _NT + s

                @pl.when(j < chunks8)
                def _():
                    pltpu.sync_copy(acc.at[pl.ds(j * 8, 8)],
                                    out_hbm.at[pl.ds(rowbase + j * 8, 8)])
                return 0

            lax.fori_loop(0, nflush, flush_body, 0)
            plsc.subcore_barrier()

    return spmm


def _pad_half(x, ehp, fill):
    pad = ehp - x.shape[0]
    return jnp.concatenate([x, jnp.full((pad,), fill, x.dtype)])


def _prep_edges(src, dst, w):
    e = src.shape[0]
    eh = e // 2
    ehp = ((eh + _EDGE_ALIGN - 1) // _EDGE_ALIGN) * _EDGE_ALIGN
    src = src.astype(jnp.int32)
    dst = dst.astype(jnp.int32)
    w = w.astype(jnp.float32)
    src2 = jnp.stack([_pad_half(src[:eh], ehp, 0), _pad_half(src[eh:], ehp, 0)])
    dst2 = jnp.stack([_pad_half(dst[:eh], ehp, 0), _pad_half(dst[eh:], ehp, 0)])
    w2 = jnp.stack([_pad_half(w[:eh], ehp, 0.0), _pad_half(w[eh:], ehp, 0.0)])
    return src2, dst2, w2, ehp


def _propagate(A, B, src, dst, w):
    nA, nB = A.shape[0], B.shape[0]
    f0 = jnp.concatenate([A, B], axis=0)
    src2, dst2, w2, ehp = _prep_edges(src, dst, w)
    zeros = jnp.zeros((_ZROWS, D), jnp.float32)
    spmm = _make_spmm(nA, nB, ehp)
    f1raw = spmm(f0, src2, dst2, w2, zeros)
    f2raw = spmm(f1raw, src2, dst2, w2, zeros)
    agg = _combine(f0, f1raw, f2raw)
    return agg[:nA], agg[nA:]


def kernel(users_feature, items_feature, bundles_feature, ui_src, ui_dst, ui_w, bi_src, bi_dst, bi_w, ub_src, ub_dst, ub_w):
    UI_u, UI_i = _propagate(users_feature, items_feature, ui_src, ui_dst, ui_w)
    BI_b, BI_i = _propagate(bundles_feature, items_feature, bi_src, bi_dst, bi_w)
    UB_u, UB_b = _propagate(users_feature, bundles_feature, ub_src, ub_dst, ub_w)
    return (UI_u, UB_u, BI_b, BI_i, UB_b, UI_i)


# speedup vs baseline: 5.0376x; 1.6478x over previous
import functools

import jax
import jax.numpy as jnp
from jax import lax
from jax.experimental import pallas as pl
from jax.experimental.pallas import tpu as pltpu
from jax.experimental.pallas import tpu_sc as plsc

D = 64
_ROW_BLK = 2000
_C = 128
_NT = 16
_NC = 2
_EDGE_ALIGN = _NT * _C * 2
_ACC_ROWS = 25088
_ZROWS = 1600
_FCH = 200

_GDN = lax.GatherDimensionNumbers(
    offset_dims=(), collapsed_slice_dims=(0,), start_index_map=(0,))


def _bcast_lane(v16, j):
    idx = jnp.full((16, 1), j, jnp.int32)
    return lax.gather(v16, idx, _GDN, slice_sizes=(1,),
                      mode=lax.GatherScatterMode.PROMISE_IN_BOUNDS)


def _l2n(x):
    n = jnp.sqrt(jnp.sum(x * x, axis=1, keepdims=True))
    return x / jnp.maximum(n, 1e-12)


def _combine_kernel(f0_ref, f1raw_ref, f2raw_ref, out_ref):
    out_ref[...] = (f0_ref[...] + _l2n(f1raw_ref[...]) + _l2n(f2raw_ref[...])) * (1.0 / 3.0)


def _rows_spec():
    return pl.BlockSpec((_ROW_BLK, D), lambda i: (i, 0))


def _combine(f0, f1raw, f2raw):
    n = f0.shape[0]
    return pl.pallas_call(
        _combine_kernel,
        grid=(n // _ROW_BLK,),
        in_specs=[_rows_spec(), _rows_spec(), _rows_spec()],
        out_specs=_rows_spec(),
        out_shape=jax.ShapeDtypeStruct((n, D), jnp.float32),
    )(f0, f1raw, f2raw)


@functools.lru_cache(maxsize=None)
def _make_spmm(n, m, ehp):
    n_total = n + m
    per_tile = ehp // _NT
    nck = per_tile // _C
    mesh = plsc.VectorSubcoreMesh(core_axis_name="c", subcore_axis_name="s")

    @functools.partial(
        pl.kernel,
        mesh=mesh,
        compiler_params=pltpu.CompilerParams(use_tc_tiling_on_sc=False),
        out_type=jax.ShapeDtypeStruct((n_total, D), jnp.float32),
        scratch_types=[
            pltpu.VMEM((2, _C), jnp.int32),
            pltpu.VMEM((2, _C), jnp.int32),
            pltpu.VMEM((2, _C), jnp.float32),
            pltpu.VMEM((2, _C), jnp.int32),
            pltpu.VMEM((_C, D), jnp.float32),
            pltpu.VMEM((_C, D), jnp.float32),
            pltpu.VMEM_SHARED((_ACC_ROWS, D), jnp.float32),
            pltpu.SemaphoreType.DMA,
            pltpu.SemaphoreType.DMA,
            pltpu.SemaphoreType.DMA,
            pltpu.SemaphoreType.DMA,
            pltpu.SemaphoreType.DMA,
            pltpu.SemaphoreType.DMA,
        ],
    )
    def spmm(f_hbm, src2, dst2, w2, zeros_hbm, out_hbm,
             srcb, dstb, wb, idxb, rows0, rows1, acc,
             tsem0, tsem1, gsem0, gsem1, ssem0, ssem1):
        c_id = lax.axis_index("c")
        s_id = lax.axis_index("s")
        rows = (rows0, rows1)
        tsem = (tsem0, tsem1)
        gsem = (gsem0, gsem1)
        ssem = (ssem0, ssem1)

        def stage_start(h, ck, p):
            off = s_id * per_tile + ck * _C
            a = pltpu.async_copy(src2.at[h, pl.ds(off, _C)], srcb.at[p], tsem[p])
            b = pltpu.async_copy(dst2.at[h, pl.ds(off, _C)], dstb.at[p], tsem[p])
            d = pltpu.async_copy(w2.at[h, pl.ds(off, _C)], wb.at[p], tsem[p])
            return a, b, d

        def stage_drain(h, ck, p):
            off = s_id * per_tile + ck * _C
            pltpu.make_async_copy(src2.at[h, pl.ds(off, _C)], srcb.at[p], tsem[p]).wait()
            pltpu.make_async_copy(dst2.at[h, pl.ds(off, _C)], dstb.at[p], tsem[p]).wait()
            pltpu.make_async_copy(w2.at[h, pl.ds(off, _C)], wb.at[p], tsem[p]).wait()

        def gather_start(p):
            pltpu.async_copy(f_hbm.at[srcb.at[p]], rows[p], gsem[p])

        def gather_drain(p):
            pltpu.make_async_copy(f_hbm.at[srcb.at[p]], rows[p], gsem[p]).wait()

        def scatter_start(p):
            pltpu.async_copy(rows[p], acc.at[idxb.at[p]], ssem[p], add=True)

        def scatter_drain(p):
            pltpu.make_async_copy(rows[p], acc.at[idxb.at[p]], ssem[p]).wait()

        for h, r_rows, obase in ((1, n // _NC, 0), (0, m // _NC, n)):
            stripe = ((r_rows + 16 + _NT * 8 - 1) // (_NT * 8)) * 8
            pltpu.sync_copy(zeros_hbm.at[pl.ds(0, stripe)],
                            acc.at[pl.ds(s_id * stripe, stripe)])
            plsc.subcore_barrier()
            rowbase = obase + c_id * r_rows

            def scale_chunk(p, r_rows=r_rows, rowbase=rowbase):
                rp = rows[p]
                for g in range(_C // 16):
                    w16 = wb[p, pl.ds(g * 16, 16)]
                    d16 = dstb[p, pl.ds(g * 16, 16)]
                    rel = d16 - rowbase
                    ok = (rel >= 0) & (rel < r_rows)
                    idxb[p, pl.ds(g * 16, 16)] = jnp.where(ok, rel, r_rows)
                    for j in range(16):
                        i = g * 16 + j
                        wr = _bcast_lane(w16, j)
                        for q in range(4):
                            rp[i, pl.ds(q * 16, 16)] = rp[i, pl.ds(q * 16, 16)] * wr

            stage_start(h, 0, 0)
            stage_start(h, 1, 1)
            stage_drain(h, 0, 0)
            gather_start(0)

            def pair_body(k2, _, h=h, scale_chunk=scale_chunk):
                for p in (0, 1):
                    ck = k2 * 2 + p
                    gather_drain(p)
                    scale_chunk(p)
                    scatter_start(p)
                    @pl.when(ck + 2 < nck)
                    def _(h=h, ck=ck, p=p):
                        stage_start(h, ck + 2, p)
                    @pl.when(ck + 1 < nck)
                    def _(h=h, ck=ck, p=p):
                        stage_drain(h, ck + 1, 1 - p)
                        @pl.when(ck >= 1)
                        def _():
                            scatter_drain(1 - p)
                        gather_start(1 - p)
                return 0

            lax.fori_loop(0, nck // 2, pair_body, 0)
            scatter_drain(0)
            scatter_drain(1)
            plsc.subcore_barrier()

            nfc = r_rows // _FCH
            nflush = (nfc + _NT - 1) // _NT

            def flush_body(kf, _, rowbase=rowbase, nfc=nfc):
                j = kf * _NT + s_id

                @pl.when(j < nfc)
                def _():
                    pltpu.sync_copy(acc.at[pl.ds(j * _FCH, _FCH)],
                                    out_hbm.at[pl.ds(rowbase + j * _FCH, _FCH)])
                return 0

            lax.fori_loop(0, nflush, flush_body, 0)
            plsc.subcore_barrier()

    return spmm


def _pad_half(x, ehp, fill):
    pad = ehp - x.shape[0]
    return jnp.concatenate([x, jnp.full((pad,), fill, x.dtype)])


def _prep_edges(src, dst, w):
    e = src.shape[0]
    eh = e // 2
    ehp = ((eh + _EDGE_ALIGN - 1) // _EDGE_ALIGN) * _EDGE_ALIGN
    src = src.astype(jnp.int32)
    dst = dst.astype(jnp.int32)
    w = w.astype(jnp.float32)
    src2 = jnp.stack([_pad_half(src[:eh], ehp, 0), _pad_half(src[eh:], ehp, 0)])
    dst2 = jnp.stack([_pad_half(dst[:eh], ehp, 0), _pad_half(dst[eh:], ehp, 0)])
    w2 = jnp.stack([_pad_half(w[:eh], ehp, 0.0), _pad_half(w[eh:], ehp, 0.0)])
    return src2, dst2, w2, ehp


def _propagate(A, B, src, dst, w):
    nA, nB = A.shape[0], B.shape[0]
    f0 = jnp.concatenate([A, B], axis=0)
    src2, dst2, w2, ehp = _prep_edges(src, dst, w)
    zeros = jnp.zeros((_ZROWS, D), jnp.float32)
    spmm = _make_spmm(nA, nB, ehp)
    f1raw = spmm(f0, src2, dst2, w2, zeros)
    f2raw = spmm(f1raw, src2, dst2, w2, zeros)
    agg = _combine(f0, f1raw, f2raw)
    return agg[:nA], agg[nA:]


def kernel(users_feature, items_feature, bundles_feature, ui_src, ui_dst, ui_w, bi_src, bi_dst, bi_w, ub_src, ub_dst, ub_w):
    UI_u, UI_i = _propagate(users_feature, items_feature, ui_src, ui_dst, ui_w)
    BI_b, BI_i = _propagate(bundles_feature, items_feature, bi_src, bi_dst, bi_w)
    UB_u, UB_b = _propagate(users_feature, bundles_feature, ub_src, ub_dst, ub_w)
    return (UI_u, UB_u, BI_b, BI_i, UB_b, UI_i)


# speedup vs baseline: 6.1345x; 1.2178x over previous
import functools

import jax
import jax.numpy as jnp
from jax import lax
from jax.experimental import pallas as pl
from jax.experimental.pallas import tpu as pltpu
from jax.experimental.pallas import tpu_sc as plsc

D = 64
_ROW_BLK = 2000
_C = 128
_NT = 16
_NC = 2
_EDGE_ALIGN = _NT * _C * 2
_HD = 32
_ACC_ROWS = 50048
_ZROWS = 3200
_FCH = 200

_GDN = lax.GatherDimensionNumbers(
    offset_dims=(), collapsed_slice_dims=(0,), start_index_map=(0,))


def _bcast_lane(v16, j):
    idx = jnp.full((16, 1), j, jnp.int32)
    return lax.gather(v16, idx, _GDN, slice_sizes=(1,),
                      mode=lax.GatherScatterMode.PROMISE_IN_BOUNDS)


def _l2n(x):
    n = jnp.sqrt(jnp.sum(x * x, axis=1, keepdims=True))
    return x / jnp.maximum(n, 1e-12)


def _combine_kernel(f0_ref, f1raw_ref, f2raw_ref, out_ref):
    out_ref[...] = (f0_ref[...] + _l2n(f1raw_ref[...]) + _l2n(f2raw_ref[...])) * (1.0 / 3.0)


def _rows_spec():
    return pl.BlockSpec((_ROW_BLK, D), lambda i: (i, 0))


def _combine(f0, f1raw, f2raw):
    n = f0.shape[0]
    return pl.pallas_call(
        _combine_kernel,
        grid=(n // _ROW_BLK,),
        in_specs=[_rows_spec(), _rows_spec(), _rows_spec()],
        out_specs=_rows_spec(),
        out_shape=jax.ShapeDtypeStruct((n, D), jnp.float32),
    )(f0, f1raw, f2raw)


@functools.lru_cache(maxsize=None)
def _make_spmm(n, m, ehp):
    n_total = n + m
    per_tile = ehp // _NT
    nck = per_tile // _C
    mesh = plsc.VectorSubcoreMesh(core_axis_name="c", subcore_axis_name="s")

    @functools.partial(
        pl.kernel,
        mesh=mesh,
        compiler_params=pltpu.CompilerParams(use_tc_tiling_on_sc=False),
        out_type=jax.ShapeDtypeStruct((_NC, n_total, _HD), jnp.float32),
        scratch_types=[
            pltpu.VMEM((2, _C), jnp.int32),
            pltpu.VMEM((2, _C), jnp.int32),
            pltpu.VMEM((2, _C), jnp.float32),
            pltpu.VMEM((2, _C), jnp.int32),
            pltpu.VMEM((_C, _HD), jnp.float32),
            pltpu.VMEM((_C, _HD), jnp.float32),
            pltpu.VMEM_SHARED((_ACC_ROWS, _HD), jnp.float32),
            pltpu.SemaphoreType.DMA,
            pltpu.SemaphoreType.DMA,
            pltpu.SemaphoreType.DMA,
            pltpu.SemaphoreType.DMA,
            pltpu.SemaphoreType.DMA,
            pltpu.SemaphoreType.DMA,
        ],
    )
    def spmm(f_hbm, src2, dst2, w2, zeros_hbm, out_hbm,
             srcb, dstb, wb, idxb, rows0, rows1, acc,
             tsem0, tsem1, gsem0, gsem1, ssem0, ssem1):
        c_id = lax.axis_index("c")
        s_id = lax.axis_index("s")
        c_off = c_id * n_total
        rows = (rows0, rows1)
        tsem = (tsem0, tsem1)
        gsem = (gsem0, gsem1)
        ssem = (ssem0, ssem1)

        def stage_start(h, ck, p):
            off = s_id * per_tile + ck * _C
            pltpu.async_copy(src2.at[h, pl.ds(off, _C)], srcb.at[p], tsem[p])
            pltpu.async_copy(dst2.at[h, pl.ds(off, _C)], dstb.at[p], tsem[p])
            pltpu.async_copy(w2.at[h, pl.ds(off, _C)], wb.at[p], tsem[p])

        def stage_drain(h, ck, p):
            off = s_id * per_tile + ck * _C
            pltpu.make_async_copy(src2.at[h, pl.ds(off, _C)], srcb.at[p], tsem[p]).wait()
            pltpu.make_async_copy(dst2.at[h, pl.ds(off, _C)], dstb.at[p], tsem[p]).wait()
            pltpu.make_async_copy(w2.at[h, pl.ds(off, _C)], wb.at[p], tsem[p]).wait()
            for g in range(_C // 16):
                sl = pl.ds(g * 16, 16)
                srcb[p, sl] = srcb[p, sl] + c_off

        def gather_start(p):
            pltpu.async_copy(f_hbm.at[srcb.at[p]], rows[p], gsem[p])

        def gather_drain(p):
            pltpu.make_async_copy(f_hbm.at[srcb.at[p]], rows[p], gsem[p]).wait()

        def scatter_start(p):
            pltpu.async_copy(rows[p], acc.at[idxb.at[p]], ssem[p], add=True)

        def scatter_drain(p):
            pltpu.make_async_copy(rows[p], acc.at[idxb.at[p]], ssem[p]).wait()

        for h, r_rows, obase in ((1, n, 0), (0, m, n)):
            stripe = ((r_rows + _NT * 8 - 1) // (_NT * 8)) * 8
            pltpu.sync_copy(zeros_hbm.at[pl.ds(0, stripe)],
                            acc.at[pl.ds(s_id * stripe, stripe)])
            plsc.subcore_barrier()

            def scale_chunk(p, obase=obase):
                rp = rows[p]
                for g in range(_C // 16):
                    w16 = wb[p, pl.ds(g * 16, 16)]
                    d16 = dstb[p, pl.ds(g * 16, 16)]
                    idxb[p, pl.ds(g * 16, 16)] = jnp.maximum(d16 - obase, 0)
                    for j in range(16):
                        i = g * 16 + j
                        wr = _bcast_lane(w16, j)
                        for q in range(_HD // 16):
                            rp[i, pl.ds(q * 16, 16)] = rp[i, pl.ds(q * 16, 16)] * wr

            stage_start(h, 0, 0)
            stage_start(h, 1, 1)
            stage_drain(h, 0, 0)
            gather_start(0)

            def pair_body(k2, _, h=h, scale_chunk=scale_chunk):
                for p in (0, 1):
                    ck = k2 * 2 + p
                    gather_drain(p)
                    scale_chunk(p)
                    scatter_start(p)
                    @pl.when(ck + 2 < nck)
                    def _(h=h, ck=ck, p=p):
                        stage_start(h, ck + 2, p)
                    @pl.when(ck + 1 < nck)
                    def _(h=h, ck=ck, p=p):
                        stage_drain(h, ck + 1, 1 - p)
                        @pl.when(ck >= 1)
                        def _():
                            scatter_drain(1 - p)
                        gather_start(1 - p)
                return 0

            lax.fori_loop(0, nck // 2, pair_body, 0)
            scatter_drain(0)
            scatter_drain(1)
            plsc.subcore_barrier()

            nfc = r_rows // _FCH
            nflush = (nfc + _NT - 1) // _NT

            def flush_body(kf, _, obase=obase, nfc=nfc):
                j = kf * _NT + s_id

                @pl.when(j < nfc)
                def _():
                    pltpu.sync_copy(acc.at[pl.ds(j * _FCH, _FCH)],
                                    out_hbm.at[c_id, pl.ds(obase + j * _FCH, _FCH)])
                return 0

            lax.fori_loop(0, nflush, flush_body, 0)
            plsc.subcore_barrier()

    return spmm


def _pad_half(x, ehp, fill):
    pad = ehp - x.shape[0]
    return jnp.concatenate([x, jnp.full((pad,), fill, x.dtype)])


def _prep_edges(src, dst, w):
    e = src.shape[0]
    eh = e // 2
    ehp = ((eh + _EDGE_ALIGN - 1) // _EDGE_ALIGN) * _EDGE_ALIGN
    src = src.astype(jnp.int32)
    dst = dst.astype(jnp.int32)
    w = w.astype(jnp.float32)
    src2 = jnp.stack([_pad_half(src[:eh], ehp, 0), _pad_half(src[eh:], ehp, 0)])
    dst2 = jnp.stack([_pad_half(dst[:eh], ehp, 0), _pad_half(dst[eh:], ehp, 0)])
    w2 = jnp.stack([_pad_half(w[:eh], ehp, 0.0), _pad_half(w[eh:], ehp, 0.0)])
    return src2, dst2, w2, ehp


def _propagate(A, B, src, dst, w):
    nA, nB = A.shape[0], B.shape[0]
    f0 = jnp.concatenate([A, B], axis=0)
    f0col = jnp.concatenate([f0[:, :_HD], f0[:, _HD:]], axis=0)
    src2, dst2, w2, ehp = _prep_edges(src, dst, w)
    zeros = jnp.zeros((_ZROWS, _HD), jnp.float32)
    spmm = _make_spmm(nA, nB, ehp)
    f1pair = spmm(f0col, src2, dst2, w2, zeros)
    f2pair = spmm(f1pair.reshape(2 * (nA + nB), _HD), src2, dst2, w2, zeros)
    f1raw = jnp.concatenate([f1pair[0], f1pair[1]], axis=1)
    f2raw = jnp.concatenate([f2pair[0], f2pair[1]], axis=1)
    agg = _combine(f0, f1raw, f2raw)
    return agg[:nA], agg[nA:]


def kernel(users_feature, items_feature, bundles_feature, ui_src, ui_dst, ui_w, bi_src, bi_dst, bi_w, ub_src, ub_dst, ub_w):
    UI_u, UI_i = _propagate(users_feature, items_feature, ui_src, ui_dst, ui_w)
    BI_b, BI_i = _propagate(bundles_feature, items_feature, bi_src, bi_dst, bi_w)
    UB_u, UB_b = _propagate(users_feature, bundles_feature, ub_src, ub_dst, ub_w)
    return (UI_u, UB_u, BI_b, BI_i, UB_b, UI_i)


# speedup vs baseline: 7.5758x; 1.2349x over previous
import functools

import jax
import jax.numpy as jnp
from jax import lax
from jax.experimental import pallas as pl
from jax.experimental.pallas import tpu as pltpu
from jax.experimental.pallas import tpu_sc as plsc

D = 64
_ROW_BLK = 2000
_C = 128
_NT = 16
_NC = 2
_NB = 4
_EDGE_ALIGN = _NT * _C * _NB
_HD = 32
_ACC_ROWS = 50048
_ZROWS = 3200
_FCH = 200

_GDN = lax.GatherDimensionNumbers(
    offset_dims=(), collapsed_slice_dims=(0,), start_index_map=(0,))


def _bcast_lane(v16, j):
    idx = jnp.full((16, 1), j, jnp.int32)
    return lax.gather(v16, idx, _GDN, slice_sizes=(1,),
                      mode=lax.GatherScatterMode.PROMISE_IN_BOUNDS)


def _l2n(x):
    n = jnp.sqrt(jnp.sum(x * x, axis=1, keepdims=True))
    return x / jnp.maximum(n, 1e-12)


def _combine_kernel(f0_ref, f1raw_ref, f2raw_ref, out_ref):
    out_ref[...] = (f0_ref[...] + _l2n(f1raw_ref[...]) + _l2n(f2raw_ref[...])) * (1.0 / 3.0)


def _rows_spec():
    return pl.BlockSpec((_ROW_BLK, D), lambda i: (i, 0))


def _combine(f0, f1raw, f2raw):
    n = f0.shape[0]
    return pl.pallas_call(
        _combine_kernel,
        grid=(n // _ROW_BLK,),
        in_specs=[_rows_spec(), _rows_spec(), _rows_spec()],
        out_specs=_rows_spec(),
        out_shape=jax.ShapeDtypeStruct((n, D), jnp.float32),
    )(f0, f1raw, f2raw)


@functools.lru_cache(maxsize=None)
def _make_spmm(n, m, ehp):
    n_total = n + m
    per_tile = ehp // _NT
    nck = per_tile // _C
    mesh = plsc.VectorSubcoreMesh(core_axis_name="c", subcore_axis_name="s")

    @functools.partial(
        pl.kernel,
        mesh=mesh,
        compiler_params=pltpu.CompilerParams(use_tc_tiling_on_sc=False),
        out_type=jax.ShapeDtypeStruct((_NC, n_total, _HD), jnp.float32),
        scratch_types=[
            pltpu.VMEM((_NB, _C), jnp.int32),
            pltpu.VMEM((_NB, _C), jnp.int32),
            pltpu.VMEM((_NB, _C), jnp.float32),
            pltpu.VMEM((_NB, _C), jnp.int32),
        ] + [pltpu.VMEM((_C, _HD), jnp.float32) for _ in range(_NB)]
          + [pltpu.VMEM_SHARED((_ACC_ROWS, _HD), jnp.float32)]
          + [pltpu.SemaphoreType.DMA for _ in range(3 * _NB)],
    )
    def spmm(f_hbm, src2, dst2, w2, zeros_hbm, out_hbm,
             srcb, dstb, wb, idxb, rows0, rows1, rows2, rows3, acc,
             tsem0, tsem1, tsem2, tsem3, gsem0, gsem1, gsem2, gsem3,
             ssem0, ssem1, ssem2, ssem3):
        c_id = lax.axis_index("c")
        s_id = lax.axis_index("s")
        c_off = c_id * n_total
        rows = (rows0, rows1, rows2, rows3)
        tsem = (tsem0, tsem1, tsem2, tsem3)
        gsem = (gsem0, gsem1, gsem2, gsem3)
        ssem = (ssem0, ssem1, ssem2, ssem3)

        def stage_start(h, ck, p):
            off = s_id * per_tile + ck * _C
            pltpu.async_copy(src2.at[h, pl.ds(off, _C)], srcb.at[p], tsem[p])
            pltpu.async_copy(dst2.at[h, pl.ds(off, _C)], dstb.at[p], tsem[p])
            pltpu.async_copy(w2.at[h, pl.ds(off, _C)], wb.at[p], tsem[p])

        def stage_drain(h, ck, p):
            off = s_id * per_tile + ck * _C
            pltpu.make_async_copy(src2.at[h, pl.ds(off, _C)], srcb.at[p], tsem[p]).wait()
            pltpu.make_async_copy(dst2.at[h, pl.ds(off, _C)], dstb.at[p], tsem[p]).wait()
            pltpu.make_async_copy(w2.at[h, pl.ds(off, _C)], wb.at[p], tsem[p]).wait()
            for g in range(_C // 16):
                sl = pl.ds(g * 16, 16)
                srcb[p, sl] = srcb[p, sl] + c_off

        def gather_start(p):
            pltpu.async_copy(f_hbm.at[srcb.at[p]], rows[p], gsem[p])

        def gather_drain(p):
            pltpu.make_async_copy(f_hbm.at[srcb.at[p]], rows[p], gsem[p]).wait()

        def scatter_start(p):
            pltpu.async_copy(rows[p], acc.at[idxb.at[p]], ssem[p], add=True)

        def scatter_drain(p):
            pltpu.make_async_copy(rows[p], acc.at[idxb.at[p]], ssem[p]).wait()

        for h, r_rows, obase in ((1, n, 0), (0, m, n)):
            stripe = ((r_rows + _NT * 8 - 1) // (_NT * 8)) * 8
            pltpu.sync_copy(zeros_hbm.at[pl.ds(0, stripe)],
                            acc.at[pl.ds(s_id * stripe, stripe)])
            plsc.subcore_barrier()

            def scale_chunk(p, obase=obase):
                rp = rows[p]
                for g in range(_C // 16):
                    w16 = wb[p, pl.ds(g * 16, 16)]
                    d16 = dstb[p, pl.ds(g * 16, 16)]
                    idxb[p, pl.ds(g * 16, 16)] = jnp.maximum(d16 - obase, 0)
                    for j in range(16):
                        i = g * 16 + j
                        wr = _bcast_lane(w16, j)
                        for q in range(_HD // 16):
                            rp[i, pl.ds(q * 16, 16)] = rp[i, pl.ds(q * 16, 16)] * wr

            for p in range(_NB):
                stage_start(h, p, p)
            stage_drain(h, 0, 0)
            gather_start(0)
            stage_drain(h, 1, 1)
            gather_start(1)

            def quad_body(k4, _, h=h, scale_chunk=scale_chunk):
                for p in range(_NB):
                    ck = k4 * _NB + p
                    gather_drain(p)
                    scale_chunk(p)
                    scatter_start(p)
                    @pl.when(ck + _NB < nck)
                    def _(h=h, ck=ck, p=p):
                        stage_start(h, ck + _NB, p)
                    @pl.when(ck + 2 < nck)
                    def _(h=h, ck=ck, p=p):
                        p2 = (p + 2) % _NB
                        stage_drain(h, ck + 2, p2)
                        @pl.when(ck >= 2)
                        def _():
                            scatter_drain(p2)
                        gather_start(p2)
                return 0

            lax.fori_loop(0, nck // _NB, quad_body, 0)
            for p in range(_NB):
                scatter_drain(p)
            plsc.subcore_barrier()

            nfc = r_rows // _FCH
            nflush = (nfc + _NT - 1) // _NT

            def flush_body(kf, _, obase=obase, nfc=nfc):
                j = kf * _NT + s_id

                @pl.when(j < nfc)
                def _():
                    pltpu.sync_copy(acc.at[pl.ds(j * _FCH, _FCH)],
                                    out_hbm.at[c_id, pl.ds(obase + j * _FCH, _FCH)])
                return 0

            lax.fori_loop(0, nflush, flush_body, 0)
            plsc.subcore_barrier()

    return spmm


def _pad_half(x, ehp, fill):
    pad = ehp - x.shape[0]
    return jnp.concatenate([x, jnp.full((pad,), fill, x.dtype)])


def _prep_edges(src, dst, w):
    e = src.shape[0]
    eh = e // 2
    ehp = ((eh + _EDGE_ALIGN - 1) // _EDGE_ALIGN) * _EDGE_ALIGN
    src = src.astype(jnp.int32)
    dst = dst.astype(jnp.int32)
    w = w.astype(jnp.float32)
    src2 = jnp.stack([_pad_half(src[:eh], ehp, 0), _pad_half(src[eh:], ehp, 0)])
    dst2 = jnp.stack([_pad_half(dst[:eh], ehp, 0), _pad_half(dst[eh:], ehp, 0)])
    w2 = jnp.stack([_pad_half(w[:eh], ehp, 0.0), _pad_half(w[eh:], ehp, 0.0)])
    return src2, dst2, w2, ehp


def _propagate(A, B, src, dst, w):
    nA, nB = A.shape[0], B.shape[0]
    f0 = jnp.concatenate([A, B], axis=0)
    f0col = jnp.concatenate([f0[:, :_HD], f0[:, _HD:]], axis=0)
    src2, dst2, w2, ehp = _prep_edges(src, dst, w)
    zeros = jnp.zeros((_ZROWS, _HD), jnp.float32)
    spmm = _make_spmm(nA, nB, ehp)
    f1pair = spmm(f0col, src2, dst2, w2, zeros)
    f2pair = spmm(f1pair.reshape(2 * (nA + nB), _HD), src2, dst2, w2, zeros)
    f1raw = jnp.concatenate([f1pair[0], f1pair[1]], axis=1)
    f2raw = jnp.concatenate([f2pair[0], f2pair[1]], axis=1)
    agg = _combine(f0, f1raw, f2raw)
    return agg[:nA], agg[nA:]


def kernel(users_feature, items_feature, bundles_feature, ui_src, ui_dst, ui_w, bi_src, bi_dst, bi_w, ub_src, ub_dst, ub_w):
    UI_u, UI_i = _propagate(users_feature, items_feature, ui_src, ui_dst, ui_w)
    BI_b, BI_i = _propagate(bundles_feature, items_feature, bi_src, bi_dst, bi_w)
    UB_u, UB_b = _propagate(users_feature, bundles_feature, ub_src, ub_dst, ub_w)
    return (UI_u, UB_u, BI_b, BI_i, UB_b, UI_i)
